# baseline (speedup 1.0000x reference)
import functools

import jax
import jax.numpy as jnp
from jax import lax
from jax.experimental import pallas as pl
from jax.experimental.pallas import tpu as pltpu
from jax.experimental.pallas import tpu_sc as plsc

N = 10000
D = 128
E = 320000
NC = 2
NS = 16
NW = NC * NS
K = 128
C = 80
C0 = 56
C1 = 104
EPAD = NW * C * K
CROWS = EPAD // K
RPS = 632
NPAD = NS * RPS
L = 16
EPS = 1e-5


def _stage1_body(h_ref, w1a_ref, w1b_ref, w2a_ref, b1_ref, b2_ref,
                 ha_ref, hb_ref, hc_ref):
    hblk = h_ref[...]
    ha_ref[...] = jnp.dot(hblk, w1a_ref[...], preferred_element_type=jnp.float32)
    hb_ref[...] = jnp.dot(hblk, w1b_ref[...], preferred_element_type=jnp.float32) + b1_ref[...]
    hc_ref[...] = jnp.dot(hblk, w2a_ref[...], preferred_element_type=jnp.float32) + b2_ref[...]


def _stage1(h2, w1a, w1b, w2a, b1, b2):
    blk = 400
    grid = (N // blk,)
    full = pl.BlockSpec((D, D), lambda i: (0, 0))
    vec = pl.BlockSpec((1, D), lambda i: (0, 0))
    row = pl.BlockSpec((blk, D), lambda i: (i, 0))
    return pl.pallas_call(
        _stage1_body,
        grid=grid,
        in_specs=[row, full, full, full, vec, vec],
        out_specs=[row, row, row],
        out_shape=[jax.ShapeDtypeStruct((N, D), jnp.float32)] * 3,
    )(h2, w1a, w1b, w2a, b1, b2)


def _rsqrt_newton(v):
    i = lax.bitcast_convert_type(v, jnp.int32)
    i = jnp.int32(0x5F3759DF) - lax.shift_right_logical(i, 1)
    y = lax.bitcast_convert_type(i, jnp.float32)
    for _ in range(2):
        y = y * (1.5 - 0.5 * v * y * y)
    return y


def _lane_sum(x):
    lanes = lax.iota(jnp.int32, L)
    for k in (1, 2, 4, 8):
        perm = lanes ^ k
        x = x + x.at[perm].get(mode="promise_in_bounds", unique_indices=True)
    return x


HK = K // 2


def _sc_edge_body(ha_hbm, hb_hbm, idx_hbm, g1_hbm, be1_hbm, zeros_hbm,
                  out_hbm,
                  idx_v, sidx0, didx0, sidx1, didx1,
                  buf_a0, buf_b0, buf_a1, buf_b1, g_v, be_v, agg_sh,
                  sa0, sb0, sa1, sb1):
    c = lax.axis_index("c")
    s = lax.axis_index("s")
    row0 = s * RPS
    base_row = jnp.where(c == 0, s * C0, NS * C0 + s * C1)

    pltpu.sync_copy(zeros_hbm.at[pl.ds(row0, RPS)], agg_sh.at[pl.ds(row0, RPS)])
    pltpu.sync_copy(idx_hbm.at[pl.ds(base_row, C1)], idx_v)
    pltpu.sync_copy(g1_hbm, g_v)
    pltpu.sync_copy(be1_hbm, be_v)
    plsc.subcore_barrier()

    g_regs = [g_v[pl.ds(L * t, L)] for t in range(D // L)]
    be_regs = [be_v[pl.ds(L * t, L)] for t in range(D // L)]

    def unpack(row, off, sidx, didx):
        for t in range(HK // L):
            p = idx_v[row, pl.ds(off + L * t, L)]
            sidx[pl.ds(L * t, L)] = lax.bitwise_and(p, jnp.int32(0xFFFF))
            didx[pl.ds(L * t, L)] = lax.shift_right_logical(p, 16)

    def start_gathers(sidx, didx, ba, bb, sa, sb):
        pltpu.async_copy(ha_hbm.at[sidx], ba, sa)
        pltpu.async_copy(hb_hbm.at[didx], bb, sb)

    def wait_gathers(sidx, didx, ba, bb, sa, sb):
        pltpu.make_async_copy(ha_hbm.at[sidx], ba, sa).wait()
        pltpu.make_async_copy(hb_hbm.at[didx], bb, sb).wait()

    def compute_scatter(ba, bb, didx):
        def edge_ln(e):
            x = [ba[e, pl.ds(L * t, L)] + bb[e, pl.ds(L * t, L)]
                 for t in range(D // L)]
            sv = ((x[0] + x[1]) + (x[2] + x[3])) + ((x[4] + x[5]) + (x[6] + x[7]))
            q = [xt * xt for xt in x]
            qv = ((q[0] + q[1]) + (q[2] + q[3])) + ((q[4] + q[5]) + (q[6] + q[7]))
            mu = _lane_sum(sv) * (1.0 / D)
            var = _lane_sum(qv) * (1.0 / D) - mu * mu
            r = _rsqrt_newton(var + EPS)
            for t in range(D // L):
                y = (x[t] - mu) * r
                y = y * g_regs[t] + be_regs[t]
                ba[e, pl.ds(L * t, L)] = jnp.maximum(y, 0.0)

        @plsc.parallel_loop(0, HK, step=1, unroll=4)
        def _edge_loop(e):
            edge_ln(e)

        pltpu.sync_copy(ba, agg_sh.at[didx], add=True)

    def run_chunks(n_chunks):
        unpack(0, 0, sidx0, didx0)
        start_gathers(sidx0, didx0, buf_a0, buf_b0, sa0, sb0)

        def pair_body(jj, carry):
            row = jj
            unpack(row, HK, sidx1, didx1)
            start_gathers(sidx1, didx1, buf_a1, buf_b1, sa1, sb1)
            wait_gathers(sidx0, didx0, buf_a0, buf_b0, sa0, sb0)
            compute_scatter(buf_a0, buf_b0, didx0)

            @pl.when(jj + 1 < n_chunks)
            def _():
                unpack(row + 1, 0, sidx0, didx0)
                start_gathers(sidx0, didx0, buf_a0, buf_b0, sa0, sb0)

            wait_gathers(sidx1, didx1, buf_a1, buf_b1, sa1, sb1)
            compute_scatter(buf_a1, buf_b1, didx1)
            return carry

        lax.fori_loop(0, n_chunks, pair_body, 0, unroll=False)

    @pl.when(c == 0)
    def _():
        run_chunks(C0)

    @pl.when(c == 1)
    def _():
        run_chunks(C1)

    plsc.subcore_barrier()
    pltpu.sync_copy(agg_sh.at[pl.ds(row0, RPS)],
                    out_hbm.at[c, pl.ds(row0, RPS)])


def _sc_edge(ha, hb, idx, g1, be1, zeros):
    mesh = plsc.VectorSubcoreMesh(core_axis_name="c", subcore_axis_name="s")
    f = functools.partial(
        pl.kernel,
        out_type=jax.ShapeDtypeStruct((NC, NPAD, D), jnp.float32),
        mesh=mesh,
        scratch_types=[
            pltpu.VMEM((C1, K), jnp.int32),
            pltpu.VMEM((HK,), jnp.int32),
            pltpu.VMEM((HK,), jnp.int32),
            pltpu.VMEM((HK,), jnp.int32),
            pltpu.VMEM((HK,), jnp.int32),
            pltpu.VMEM((HK, D), jnp.float32),
            pltpu.VMEM((HK, D), jnp.float32),
            pltpu.VMEM((HK, D), jnp.float32),
            pltpu.VMEM((HK, D), jnp.float32),
            pltpu.VMEM((D,), jnp.float32),
            pltpu.VMEM((D,), jnp.float32),
            pltpu.VMEM_SHARED((NPAD, D), jnp.float32),
            pltpu.SemaphoreType.DMA,
            pltpu.SemaphoreType.DMA,
            pltpu.SemaphoreType.DMA,
            pltpu.SemaphoreType.DMA,
        ],
    )(_sc_edge_body)
    return f(ha, hb, idx, g1, be1, zeros)


def _stage3_body(hc_ref, agg_ref, h_ref, w2b_ref, g2_ref, be2_ref, out_ref):
    aggsum = agg_ref[0] + agg_ref[1]
    t = hc_ref[...] + jnp.dot(aggsum, w2b_ref[...], preferred_element_type=jnp.float32)
    mu = jnp.mean(t, axis=-1, keepdims=True)
    var = jnp.mean((t - mu) ** 2, axis=-1, keepdims=True)
    y = (t - mu) * lax.rsqrt(var + EPS) * g2_ref[...] + be2_ref[...]
    out_ref[...] = jnp.maximum(y, 0.0) + h_ref[...]


def _stage3(hc, agg, h2, w2b, g2, be2):
    blk = 400
    grid = (N // blk,)
    row = pl.BlockSpec((blk, D), lambda i: (i, 0))
    return pl.pallas_call(
        _stage3_body,
        grid=grid,
        in_specs=[
            row,
            pl.BlockSpec((NC, blk, D), lambda i: (0, i, 0)),
            row,
            pl.BlockSpec((D, D), lambda i: (0, 0)),
            pl.BlockSpec((1, D), lambda i: (0, 0)),
            pl.BlockSpec((1, D), lambda i: (0, 0)),
        ],
        out_specs=row,
        out_shape=jax.ShapeDtypeStruct((N, D), jnp.float32),
    )(hc, agg, h2, w2b, g2, be2)


def kernel(h, edge_index, W1, b1, g1, be1, W2, b2, g2, be2):
    h2 = h.reshape(N, D)
    npad = EPAD - E
    src = jnp.concatenate([edge_index[0], jnp.zeros((npad,), jnp.int32)])
    dst = jnp.concatenate([edge_index[1], jnp.full((npad,), N, jnp.int32)])
    idx = (src | (dst << 16)).reshape(CROWS, K)
    zeros = jnp.zeros((NPAD, D), jnp.float32)
    ha, hb, hc = _stage1(h2, W1[:D], W1[D:], W2[:D],
                         b1.reshape(1, D), b2.reshape(1, D))
    agg = _sc_edge(ha, hb, idx, g1, be1, zeros)
    out = _stage3(hc, agg, h2, W2[D:], g2.reshape(1, D), be2.reshape(1, D))
    return out.reshape(1, N, D)


# speedup vs baseline: 1.2352x; 1.2352x over previous
import functools

import jax
import jax.numpy as jnp
from jax import lax
from jax.experimental import pallas as pl
from jax.experimental.pallas import tpu as pltpu
from jax.experimental.pallas import tpu_sc as plsc

N = 10000
D = 128
E = 320000
NC = 2
NS = 16
NW = NC * NS
K = 128
C = 80
C0 = 104
C1 = 56
EPAD = NW * C * K
CROWS = EPAD // K
RPS = 632
NPAD = NS * RPS
L = 16
EPS = 1e-5


def _stage1_body(h_ref, w1a_ref, w1b_ref, w2a_ref, b1_ref, b2_ref,
                 ha_ref, hb_ref, hc_ref):
    hblk = h_ref[...]
    ha_ref[...] = jnp.dot(hblk, w1a_ref[...], preferred_element_type=jnp.float32)
    hb_ref[...] = jnp.dot(hblk, w1b_ref[...], preferred_element_type=jnp.float32) + b1_ref[...]
    hc_ref[...] = jnp.dot(hblk, w2a_ref[...], preferred_element_type=jnp.float32) + b2_ref[...]


def _stage1(h2, w1a, w1b, w2a, b1, b2):
    blk = 400
    grid = (N // blk,)
    full = pl.BlockSpec((D, D), lambda i: (0, 0))
    vec = pl.BlockSpec((1, D), lambda i: (0, 0))
    row = pl.BlockSpec((blk, D), lambda i: (i, 0))
    return pl.pallas_call(
        _stage1_body,
        grid=grid,
        in_specs=[row, full, full, full, vec, vec],
        out_specs=[row, row, row],
        out_shape=[jax.ShapeDtypeStruct((N, D), jnp.float32)] * 3,
    )(h2, w1a, w1b, w2a, b1, b2)


def _rsqrt_newton(v):
    i = lax.bitcast_convert_type(v, jnp.int32)
    i = jnp.int32(0x5F3759DF) - lax.shift_right_logical(i, 1)
    y = lax.bitcast_convert_type(i, jnp.float32)
    for _ in range(2):
        y = y * (1.5 - 0.5 * v * y * y)
    return y


def _lane_sum(x):
    lanes = lax.iota(jnp.int32, L)
    for k in (1, 2, 4, 8):
        perm = lanes ^ k
        x = x + x.at[perm].get(mode="promise_in_bounds", unique_indices=True)
    return x


HK = K // 2


def _sc_edge_body(ha_hbm, hb_hbm, idx_hbm, g1_hbm, be1_hbm, zeros_hbm,
                  out_hbm,
                  idx_v, sidx0, didx0, sidx1, didx1,
                  buf_a0, buf_b0, buf_a1, buf_b1, g_v, be_v, agg_sh,
                  sa0, sb0, sa1, sb1):
    c = lax.axis_index("c")
    s = lax.axis_index("s")
    row0 = s * RPS
    base_row = jnp.where(c == 0, s * C0, NS * C0 + s * C1)

    pltpu.sync_copy(zeros_hbm.at[pl.ds(row0, RPS)], agg_sh.at[pl.ds(row0, RPS)])
    pltpu.sync_copy(g1_hbm, g_v)
    pltpu.sync_copy(be1_hbm, be_v)
    plsc.subcore_barrier()

    g_regs = [g_v[pl.ds(L * t, L)] for t in range(D // L)]
    be_regs = [be_v[pl.ds(L * t, L)] for t in range(D // L)]

    def unpack(row, off, sidx, didx):
        for t in range(HK // L):
            p = idx_v[row, pl.ds(off + L * t, L)]
            sidx[pl.ds(L * t, L)] = lax.bitwise_and(p, jnp.int32(0xFFFF))
            didx[pl.ds(L * t, L)] = lax.shift_right_logical(p, 16)

    def start_gathers(sidx, didx, ba, bb, sa, sb):
        pltpu.async_copy(ha_hbm.at[sidx], ba, sa)
        pltpu.async_copy(hb_hbm.at[didx], bb, sb)

    def wait_gathers(sidx, didx, ba, bb, sa, sb):
        pltpu.make_async_copy(ha_hbm.at[sidx], ba, sa).wait()
        pltpu.make_async_copy(hb_hbm.at[didx], bb, sb).wait()

    def compute_scatter(ba, bb, didx):
        def edge_ln(e):
            x = [ba[e, pl.ds(L * t, L)] + bb[e, pl.ds(L * t, L)]
                 for t in range(D // L)]
            sv = ((x[0] + x[1]) + (x[2] + x[3])) + ((x[4] + x[5]) + (x[6] + x[7]))
            q = [xt * xt for xt in x]
            qv = ((q[0] + q[1]) + (q[2] + q[3])) + ((q[4] + q[5]) + (q[6] + q[7]))
            mu = _lane_sum(sv) * (1.0 / D)
            var = _lane_sum(qv) * (1.0 / D) - mu * mu
            r = _rsqrt_newton(var + EPS)
            for t in range(D // L):
                y = (x[t] - mu) * r
                y = y * g_regs[t] + be_regs[t]
                ba[e, pl.ds(L * t, L)] = jnp.maximum(y, 0.0)

        @plsc.parallel_loop(0, HK, step=1, unroll=4)
        def _edge_loop(e):
            edge_ln(e)

        pltpu.sync_copy(ba, agg_sh.at[didx], add=True)

    def run_chunks(n_chunks):
        pltpu.sync_copy(idx_hbm.at[pl.ds(base_row, n_chunks)],
                        idx_v.at[pl.ds(0, n_chunks)])
        unpack(0, 0, sidx0, didx0)
        start_gathers(sidx0, didx0, buf_a0, buf_b0, sa0, sb0)

        def pair_body(jj, carry):
            row = jj
            unpack(row, HK, sidx1, didx1)
            start_gathers(sidx1, didx1, buf_a1, buf_b1, sa1, sb1)
            wait_gathers(sidx0, didx0, buf_a0, buf_b0, sa0, sb0)
            compute_scatter(buf_a0, buf_b0, didx0)

            @pl.when(jj + 1 < n_chunks)
            def _():
                unpack(row + 1, 0, sidx0, didx0)
                start_gathers(sidx0, didx0, buf_a0, buf_b0, sa0, sb0)

            wait_gathers(sidx1, didx1, buf_a1, buf_b1, sa1, sb1)
            compute_scatter(buf_a1, buf_b1, didx1)
            return carry

        lax.fori_loop(0, n_chunks, pair_body, 0, unroll=False)

    @pl.when(c == 0)
    def _():
        run_chunks(C0)

    @pl.when(c == 1)
    def _():
        run_chunks(C1)

    plsc.subcore_barrier()
    pltpu.sync_copy(agg_sh.at[pl.ds(row0, RPS)],
                    out_hbm.at[c, pl.ds(row0, RPS)])


def _sc_edge(ha, hb, idx, g1, be1, zeros):
    mesh = plsc.VectorSubcoreMesh(core_axis_name="c", subcore_axis_name="s")
    f = functools.partial(
        pl.kernel,
        out_type=jax.ShapeDtypeStruct((NC, NPAD, D), jnp.float32),
        mesh=mesh,
        scratch_types=[
            pltpu.VMEM((max(C0, C1), K), jnp.int32),
            pltpu.VMEM((HK,), jnp.int32),
            pltpu.VMEM((HK,), jnp.int32),
            pltpu.VMEM((HK,), jnp.int32),
            pltpu.VMEM((HK,), jnp.int32),
            pltpu.VMEM((HK, D), jnp.float32),
            pltpu.VMEM((HK, D), jnp.float32),
            pltpu.VMEM((HK, D), jnp.float32),
            pltpu.VMEM((HK, D), jnp.float32),
            pltpu.VMEM((D,), jnp.float32),
            pltpu.VMEM((D,), jnp.float32),
            pltpu.VMEM_SHARED((NPAD, D), jnp.float32),
            pltpu.SemaphoreType.DMA,
            pltpu.SemaphoreType.DMA,
            pltpu.SemaphoreType.DMA,
            pltpu.SemaphoreType.DMA,
        ],
    )(_sc_edge_body)
    return f(ha, hb, idx, g1, be1, zeros)


def _stage3_body(hc_ref, agg_ref, h_ref, w2b_ref, g2_ref, be2_ref, out_ref):
    aggsum = agg_ref[0] + agg_ref[1]
    t = hc_ref[...] + jnp.dot(aggsum, w2b_ref[...], preferred_element_type=jnp.float32)
    mu = jnp.mean(t, axis=-1, keepdims=True)
    var = jnp.mean((t - mu) ** 2, axis=-1, keepdims=True)
    y = (t - mu) * lax.rsqrt(var + EPS) * g2_ref[...] + be2_ref[...]
    out_ref[...] = jnp.maximum(y, 0.0) + h_ref[...]


def _stage3(hc, agg, h2, w2b, g2, be2):
    blk = 400
    grid = (N // blk,)
    row = pl.BlockSpec((blk, D), lambda i: (i, 0))
    return pl.pallas_call(
        _stage3_body,
        grid=grid,
        in_specs=[
            row,
            pl.BlockSpec((NC, blk, D), lambda i: (0, i, 0)),
            row,
            pl.BlockSpec((D, D), lambda i: (0, 0)),
            pl.BlockSpec((1, D), lambda i: (0, 0)),
            pl.BlockSpec((1, D), lambda i: (0, 0)),
        ],
        out_specs=row,
        out_shape=jax.ShapeDtypeStruct((N, D), jnp.float32),
    )(hc, agg, h2, w2b, g2, be2)


def kernel(h, edge_index, W1, b1, g1, be1, W2, b2, g2, be2):
    h2 = h.reshape(N, D)
    npad = EPAD - E
    src = jnp.concatenate([edge_index[0], jnp.zeros((npad,), jnp.int32)])
    dst = jnp.concatenate([edge_index[1], jnp.full((npad,), N, jnp.int32)])
    idx = (src | (dst << 16)).reshape(CROWS, K)
    zeros = jnp.zeros((NPAD, D), jnp.float32)
    ha, hb, hc = _stage1(h2, W1[:D], W1[D:], W2[:D],
                         b1.reshape(1, D), b2.reshape(1, D))
    agg = _sc_edge(ha, hb, idx, g1, be1, zeros)
    out = _stage3(hc, agg, h2, W2[D:], g2.reshape(1, D), be2.reshape(1, D))
    return out.reshape(1, N, D)
